# baseline (speedup 1.0000x reference)
import functools

import jax
import jax.numpy as jnp
from jax import lax
from jax.experimental import pallas as pl
from jax.experimental.pallas import tpu as pltpu
from jax.experimental.pallas import tpu_sc as plsc

DIM = 128
GAMMA = 12.0
BATCH = 16384

NC = 2
NS = 16
L = 16
NW = NC * NS
B_PER_W = BATCH // NW
CH = 32
N_CH = B_PER_W // CH
NBUF = 3
KSEG = DIM // L


@functools.cache
def _build():
    mesh = plsc.VectorSubcoreMesh(
        core_axis_name="c", subcore_axis_name="s",
        num_cores=NC, num_subcores=NS,
    )

    @functools.partial(
        pl.kernel,
        mesh=mesh,
        compiler_params=pltpu.CompilerParams(needs_layout_passes=False),
        out_type=jax.ShapeDtypeStruct((BATCH,), jnp.float32),
        scratch_types=[
            pltpu.VMEM((B_PER_W,), jnp.int32),
            pltpu.VMEM((B_PER_W,), jnp.int32),
            pltpu.VMEM((B_PER_W,), jnp.int32),
            pltpu.VMEM((NBUF, CH, DIM), jnp.float32),
            pltpu.VMEM((NBUF, CH, DIM), jnp.float32),
            pltpu.VMEM((NBUF, CH, DIM), jnp.float32),
            pltpu.VMEM((B_PER_W,), jnp.float32),
            pltpu.SemaphoreType.DMA((NBUF,)),
        ],
    )
    def transe_kernel(hidx_hbm, ridx_hbm, tidx_hbm, ent_hbm, rel_hbm,
                      out_hbm, hi_v, ri_v, ti_v, h_v, r_v, t_v,
                      o_v, sem):
        w = lax.axis_index("s") * NC + lax.axis_index("c")
        base = w * B_PER_W

        bsl = pl.ds(base, B_PER_W)
        cp_h = pltpu.async_copy(hidx_hbm.at[bsl], hi_v, sem.at[0])
        cp_r = pltpu.async_copy(ridx_hbm.at[bsl], ri_v, sem.at[1])
        cp_t = pltpu.async_copy(tidx_hbm.at[bsl], ti_v, sem.at[2])
        cp_h.wait()
        cp_r.wait()
        cp_t.wait()

        lane = lax.iota(jnp.int32, L)

        def fire(j, b):
            sl = pl.ds(j * CH, CH)
            pltpu.async_copy(ent_hbm.at[hi_v.at[sl]], h_v.at[b], sem.at[b])
            pltpu.async_copy(rel_hbm.at[ri_v.at[sl]], r_v.at[b], sem.at[b])
            pltpu.async_copy(ent_hbm.at[ti_v.at[sl]], t_v.at[b], sem.at[b])

        def drain(j, b):
            sl = pl.ds(j * CH, CH)
            pltpu.make_async_copy(ent_hbm.at[hi_v.at[sl]], h_v.at[b],
                                  sem.at[b]).wait()
            pltpu.make_async_copy(rel_hbm.at[ri_v.at[sl]], r_v.at[b],
                                  sem.at[b]).wait()
            pltpu.make_async_copy(ent_hbm.at[ti_v.at[sl]], t_v.at[b],
                                  sem.at[b]).wait()

        def compute(j, b):
            hb, rb, tb = h_v.at[b], r_v.at[b], t_v.at[b]

            @plsc.parallel_loop(0, CH // L, 1)
            def _group(g):
                zero = jnp.zeros((L,), jnp.float32)

                @plsc.parallel_loop(0, L, 1, unroll=2, carry=zero)
                def svec(i, sv):
                    row = g * L + i
                    acc0 = jnp.zeros((L,), jnp.float32)
                    acc1 = jnp.zeros((L,), jnp.float32)
                    for k in range(0, KSEG, 2):
                        hv = hb[row, pl.ds(k * L, L)]
                        rv = rb[row, pl.ds(k * L, L)]
                        tv = tb[row, pl.ds(k * L, L)]
                        acc0 = acc0 + jnp.abs(hv + rv - tv)
                        hv = hb[row, pl.ds((k + 1) * L, L)]
                        rv = rb[row, pl.ds((k + 1) * L, L)]
                        tv = tb[row, pl.ds((k + 1) * L, L)]
                        acc1 = acc1 + jnp.abs(hv + rv - tv)
                    s = jnp.sum(acc0 + acc1)
                    return sv + jnp.where(lane == i, s, 0.0)

                o_v[pl.ds(j * CH + g * L, L)] = GAMMA - svec

        fire(0, 0)
        fire(1, 1)

        def do_chunk(j, carry):
            b = j % NBUF
            drain(j, b)

            @pl.when(j + 2 < N_CH)
            def _():
                fire(j + 2, (j + 2) % NBUF)

            compute(j, b)
            return carry

        lax.fori_loop(0, N_CH, do_chunk, 0)
        pltpu.sync_copy(o_v, out_hbm.at[pl.ds(base, B_PER_W)])

    return transe_kernel


def kernel(sample, entity_embedding, relation_embedding):
    hidx = sample[:, 0]
    ridx = sample[:, 1]
    tidx = sample[:, 2]
    score = _build()(hidx, ridx, tidx, entity_embedding, relation_embedding)
    return score.reshape(BATCH, 1)


# speedup vs baseline: 1.0194x; 1.0194x over previous
import functools

import jax
import jax.numpy as jnp
from jax import lax
from jax.experimental import pallas as pl
from jax.experimental.pallas import tpu as pltpu
from jax.experimental.pallas import tpu_sc as plsc

DIM = 128
GAMMA = 12.0
BATCH = 16384

NC = 2
NS = 16
L = 16
NW = NC * NS
B_PER_W = BATCH // NW
CH = 64
N_CH = B_PER_W // CH
NBUF = 3
KSEG = DIM // L


@functools.cache
def _build():
    mesh = plsc.VectorSubcoreMesh(
        core_axis_name="c", subcore_axis_name="s",
        num_cores=NC, num_subcores=NS,
    )

    @functools.partial(
        pl.kernel,
        mesh=mesh,
        compiler_params=pltpu.CompilerParams(needs_layout_passes=False),
        out_type=jax.ShapeDtypeStruct((BATCH,), jnp.float32),
        scratch_types=[
            pltpu.VMEM((B_PER_W,), jnp.int32),
            pltpu.VMEM((B_PER_W,), jnp.int32),
            pltpu.VMEM((B_PER_W,), jnp.int32),
            pltpu.VMEM((NBUF, CH, DIM), jnp.float32),
            pltpu.VMEM((NBUF, CH, DIM), jnp.float32),
            pltpu.VMEM((NBUF, CH, DIM), jnp.float32),
            pltpu.VMEM((B_PER_W,), jnp.float32),
            pltpu.SemaphoreType.DMA((NBUF,)),
        ],
    )
    def transe_kernel(hidx_hbm, ridx_hbm, tidx_hbm, ent_hbm, rel_hbm,
                      out_hbm, hi_v, ri_v, ti_v, h_v, r_v, t_v,
                      o_v, sem):
        w = lax.axis_index("s") * NC + lax.axis_index("c")
        base = w * B_PER_W

        bsl = pl.ds(base, B_PER_W)
        cp_h = pltpu.async_copy(hidx_hbm.at[bsl], hi_v, sem.at[0])
        cp_r = pltpu.async_copy(ridx_hbm.at[bsl], ri_v, sem.at[1])
        cp_t = pltpu.async_copy(tidx_hbm.at[bsl], ti_v, sem.at[2])
        cp_h.wait()
        cp_r.wait()
        cp_t.wait()

        lane = lax.iota(jnp.int32, L)

        def fire(j, b):
            sl = pl.ds(j * CH, CH)
            pltpu.async_copy(ent_hbm.at[hi_v.at[sl]], h_v.at[b], sem.at[b])
            pltpu.async_copy(rel_hbm.at[ri_v.at[sl]], r_v.at[b], sem.at[b])
            pltpu.async_copy(ent_hbm.at[ti_v.at[sl]], t_v.at[b], sem.at[b])

        def drain(j, b):
            sl = pl.ds(j * CH, CH)
            pltpu.make_async_copy(ent_hbm.at[hi_v.at[sl]], h_v.at[b],
                                  sem.at[b]).wait()
            pltpu.make_async_copy(rel_hbm.at[ri_v.at[sl]], r_v.at[b],
                                  sem.at[b]).wait()
            pltpu.make_async_copy(ent_hbm.at[ti_v.at[sl]], t_v.at[b],
                                  sem.at[b]).wait()

        def compute(j, b):
            hb, rb, tb = h_v.at[b], r_v.at[b], t_v.at[b]

            @plsc.parallel_loop(0, CH // L, 1)
            def _group(g):
                zero = jnp.zeros((L,), jnp.float32)

                @plsc.parallel_loop(0, L, 1, unroll=1, carry=zero)
                def svec(i, sv):
                    row = g * L + i
                    acc0 = jnp.zeros((L,), jnp.float32)
                    acc1 = jnp.zeros((L,), jnp.float32)
                    for k in range(0, KSEG, 2):
                        hv = hb[row, pl.ds(k * L, L)]
                        rv = rb[row, pl.ds(k * L, L)]
                        tv = tb[row, pl.ds(k * L, L)]
                        acc0 = acc0 + jnp.abs(hv + rv - tv)
                        hv = hb[row, pl.ds((k + 1) * L, L)]
                        rv = rb[row, pl.ds((k + 1) * L, L)]
                        tv = tb[row, pl.ds((k + 1) * L, L)]
                        acc1 = acc1 + jnp.abs(hv + rv - tv)
                    s = jnp.sum(acc0 + acc1)
                    return sv + jnp.where(lane == i, s, 0.0)

                o_v[pl.ds(j * CH + g * L, L)] = GAMMA - svec

        fire(0, 0)
        fire(1, 1)

        def do_chunk(j, carry):
            b = j % NBUF
            drain(j, b)

            @pl.when(j + 2 < N_CH)
            def _():
                fire(j + 2, (j + 2) % NBUF)

            compute(j, b)
            return carry

        lax.fori_loop(0, N_CH, do_chunk, 0)
        pltpu.sync_copy(o_v, out_hbm.at[pl.ds(base, B_PER_W)])

    return transe_kernel


def kernel(sample, entity_embedding, relation_embedding):
    hidx = sample[:, 0]
    ridx = sample[:, 1]
    tidx = sample[:, 2]
    score = _build()(hidx, ridx, tidx, entity_embedding, relation_embedding)
    return score.reshape(BATCH, 1)
